# baseline (speedup 1.0000x reference)
import functools

import jax
import jax.numpy as jnp
from jax import lax
from jax.experimental import pallas as pl
from jax.experimental.pallas import tpu as pltpu
from jax.experimental.pallas import tpu_sc as plsc

B, L, V, D, C = 16384, 62, 100000, 50, 5
DP = 64
NC, NS, LANES = 2, 16, 16
NW = NC * NS
SPW = B // NW


GROUPS = SPW // LANES


CHUNKS = (16, 16, 16, 14)
NBUF = 4


def _sc_pool_body(x_hbm, l_hbm, table_hbm, out_hbm,
                  x_v, l_v, rep_v, bufs, sems):
    wid = lax.axis_index("s") * NC + lax.axis_index("c")
    base = wid * SPW
    pltpu.sync_copy(x_hbm.at[pl.ds(base, SPW)], x_v)
    pltpu.sync_copy(l_hbm.at[pl.ds(base, SPW)], l_v.at[pl.ds(0, SPW)])

    def issue(s, n, b):
        for c, sz in enumerate(CHUNKS):
            idx = x_v.at[s, pl.ds(16 * c, sz)]
            dst = bufs[b].at[pl.ds(16 * c, sz)]
            if c == 0:
                pltpu.async_copy(table_hbm.at[idx], dst, sems[b])
            else:
                @pl.when(16 * c < n)
                def _():
                    pltpu.async_copy(table_hbm.at[idx], dst, sems[b])

    def drain(n, b):
        for c, sz in enumerate(CHUNKS):
            dst = bufs[b].at[pl.ds(16 * c, sz)]
            cp = pltpu.make_async_copy(table_hbm.at[x_v.at[0, pl.ds(0, sz)]],
                                       dst, sems[b])
            if c == 0:
                cp.wait()
            else:
                @pl.when(16 * c < n)
                def _(cp=cp):
                    cp.wait()

    def accumulate(s, n, inv, buf):
        def inner(j, acc):
            a0, a1, a2, a3 = acc
            e0, o0 = plsc.unpack(buf[j, pl.ds(0, 2 * LANES)],
                                 format=plsc.PackFormat.INTERLEAVED,
                                 preferred_element_type=jnp.float32)
            e1, o1 = plsc.unpack(buf[j, pl.ds(2 * LANES, 2 * LANES)],
                                 format=plsc.PackFormat.INTERLEAVED,
                                 preferred_element_type=jnp.float32)
            return (a0 + e0, a1 + o0, a2 + e1, a3 + o1)

        zero = jnp.zeros((LANES,), jnp.float32)
        a0, a1, a2, a3 = lax.fori_loop(0, n, inner, (zero, zero, zero, zero))
        rep_v[s, pl.ds(0, LANES)] = a0 * inv
        rep_v[s, pl.ds(LANES, LANES)] = a1 * inv
        rep_v[s, pl.ds(2 * LANES, LANES)] = a2 * inv
        rep_v[s, pl.ds(3 * LANES, LANES)] = a3 * inv

    nv0 = l_v[pl.ds(0, LANES)]
    for k in range(NBUF):
        issue(k, nv0[k], k)

    def group(g, carry):
        nv = l_v[pl.ds(g * LANES, LANES)]
        nvn = l_v[pl.ds(g * LANES + LANES, LANES)]
        inv_v = 1.0 / nv.astype(jnp.float32)
        last = g == GROUPS - 1
        for k in range(LANES):
            s = g * LANES + k
            b = k % NBUF
            drain(nv[k], b)
            accumulate(s, nv[k], inv_v[k], bufs[b])
            if k < LANES - NBUF:
                issue(s + NBUF, nv[k + NBUF], b)
            else:
                @pl.when(jnp.logical_not(last))
                def _(k=k, s=s, b=b):
                    issue(s + NBUF, nvn[k - (LANES - NBUF)], b)

        return carry

    lax.fori_loop(0, GROUPS, group, 0)
    pltpu.sync_copy(rep_v, out_hbm.at[pl.ds(base, SPW)])


def _sc_pool(x, l, table):
    return pl.kernel(
        _sc_pool_body,
        out_type=jax.ShapeDtypeStruct((B, DP), jnp.float32),
        mesh=plsc.VectorSubcoreMesh(core_axis_name="c", subcore_axis_name="s"),
        scratch_types=[
            pltpu.VMEM((SPW, L), jnp.int32),
            pltpu.VMEM((SPW + LANES,), jnp.int32),
            pltpu.VMEM((SPW, DP), jnp.float32),
            tuple(pltpu.VMEM((L, DP), jnp.bfloat16) for _ in range(NBUF)),
            tuple(pltpu.SemaphoreType.DMA for _ in range(NBUF)),
        ],
        compiler_params=pltpu.CompilerParams(use_tc_tiling_on_sc=False,
                                            needs_layout_passes=False),
    )(x, l, table)


CBT = 2000


def _cast_body(t_ref, o_ref):
    blk = t_ref[...].astype(jnp.bfloat16)
    o_ref[...] = jnp.concatenate(
        [blk, jnp.zeros((CBT, DP - D), jnp.bfloat16)], axis=1)


def _cast(table):
    return pl.pallas_call(
        _cast_body,
        out_shape=jax.ShapeDtypeStruct((V, DP), jnp.bfloat16),
        grid=(V // CBT,),
        in_specs=[pl.BlockSpec((CBT, D), lambda i: (i, 0))],
        out_specs=pl.BlockSpec((CBT, DP), lambda i: (i, 0)),
    )(table)


BT = 2048


def _head_body(rep_ref, wt_ref, b_ref, o_ref):
    r = jnp.tanh(rep_ref[...])
    o_ref[...] = (
        jnp.dot(r, wt_ref[...], preferred_element_type=jnp.float32)
        + b_ref[...]
    )


def _head(rep, wt, b2d):
    return pl.pallas_call(
        _head_body,
        out_shape=jax.ShapeDtypeStruct((B, C), jnp.float32),
        grid=(B // BT,),
        in_specs=[
            pl.BlockSpec((BT, DP), lambda i: (i, 0)),
            pl.BlockSpec((DP, C), lambda i: (0, 0)),
            pl.BlockSpec((1, C), lambda i: (0, 0)),
        ],
        out_specs=pl.BlockSpec((BT, C), lambda i: (i, 0)),
    )(rep, wt, b2d)


_PERM = (tuple(range(0, DP // 2, 2)) + tuple(range(1, DP // 2, 2))
         + tuple(range(DP // 2, DP, 2)) + tuple(range(DP // 2 + 1, DP, 2)))


def kernel(x, l, lengths, table, W, b):
    table_b = _cast(table)
    wt_p = jnp.pad(W.T, ((0, DP - D), (0, 0)))[_PERM, :]
    rep = _sc_pool(x, l, table_b)
    return _head(rep, wt_p, b.reshape(1, C))


# speedup vs baseline: 1.1150x; 1.1150x over previous
import jax
import jax.numpy as jnp
from jax import lax
from jax.experimental import pallas as pl
from jax.experimental.pallas import tpu as pltpu
from jax.experimental.pallas import tpu_sc as plsc

B, L, V, D, C = 16384, 62, 100000, 50, 5
DP = 64
NC, NS, LANES = 2, 16, 16
NW = NC * NS
SPW = B // NW


GROUPS = SPW // LANES


CHUNKS = (16, 16, 16, 14)
NBUF = 4


def _sc_pool_body(x_hbm, l_hbm, table_hbm, out_hbm,
                  x_v, l_v, rep_v, bufs, sems):
    wid = lax.axis_index("s") * NC + lax.axis_index("c")
    base = wid * SPW
    pltpu.sync_copy(x_hbm.at[pl.ds(base, SPW)], x_v)
    pltpu.sync_copy(l_hbm.at[pl.ds(base, SPW)], l_v.at[pl.ds(0, SPW)])

    def issue(s, n, b):
        for c, sz in enumerate(CHUNKS):
            idx = x_v.at[s, pl.ds(16 * c, sz)]
            dst = bufs[b].at[pl.ds(16 * c, sz)]
            if c == 0:
                pltpu.async_copy(table_hbm.at[idx], dst, sems[b])
            else:
                @pl.when(16 * c < n)
                def _():
                    pltpu.async_copy(table_hbm.at[idx], dst, sems[b])

    def drain(n, b):
        for c, sz in enumerate(CHUNKS):
            dst = bufs[b].at[pl.ds(16 * c, sz)]
            cp = pltpu.make_async_copy(table_hbm.at[x_v.at[0, pl.ds(0, sz)]],
                                       dst, sems[b])
            if c == 0:
                cp.wait()
            else:
                @pl.when(16 * c < n)
                def _(cp=cp):
                    cp.wait()

    def accumulate(s, n, inv, buf):
        def inner(j, acc):
            a0, a1, a2, a3 = acc
            e0, o0 = plsc.unpack(buf[j, pl.ds(0, 2 * LANES)],
                                 format=plsc.PackFormat.INTERLEAVED,
                                 preferred_element_type=jnp.float32)
            e1, o1 = plsc.unpack(buf[j, pl.ds(2 * LANES, 2 * LANES)],
                                 format=plsc.PackFormat.INTERLEAVED,
                                 preferred_element_type=jnp.float32)
            return (a0 + e0, a1 + o0, a2 + e1, a3 + o1)

        zero = jnp.zeros((LANES,), jnp.float32)
        a0, a1, a2, a3 = lax.fori_loop(0, n, inner, (zero, zero, zero, zero))
        rep_v[s, pl.ds(0, LANES)] = a0 * inv
        rep_v[s, pl.ds(LANES, LANES)] = a1 * inv
        rep_v[s, pl.ds(2 * LANES, LANES)] = a2 * inv
        rep_v[s, pl.ds(3 * LANES, LANES)] = a3 * inv

    nv0 = l_v[pl.ds(0, LANES)]
    for k in range(NBUF):
        issue(k, nv0[k], k)

    def group(g, carry):
        nv = l_v[pl.ds(g * LANES, LANES)]
        nvn = l_v[pl.ds(g * LANES + LANES, LANES)]
        inv_v = 1.0 / nv.astype(jnp.float32)
        last = g == GROUPS - 1
        for k in range(LANES):
            s = g * LANES + k
            b = k % NBUF
            drain(nv[k], b)
            accumulate(s, nv[k], inv_v[k], bufs[b])
            if k < LANES - NBUF:
                issue(s + NBUF, nv[k + NBUF], b)
            else:
                @pl.when(jnp.logical_not(last))
                def _(k=k, s=s, b=b):
                    issue(s + NBUF, nvn[k - (LANES - NBUF)], b)

        return carry

    lax.fori_loop(0, GROUPS, group, 0)
    pltpu.sync_copy(rep_v, out_hbm.at[pl.ds(base, SPW)])


def _sc_pool(x, l, table):
    return pl.kernel(
        _sc_pool_body,
        out_type=jax.ShapeDtypeStruct((B, DP), jnp.float32),
        mesh=plsc.VectorSubcoreMesh(core_axis_name="c", subcore_axis_name="s"),
        scratch_types=[
            pltpu.VMEM((SPW, L), jnp.int32),
            pltpu.VMEM((SPW + LANES,), jnp.int32),
            pltpu.VMEM((SPW, DP), jnp.float32),
            tuple(pltpu.VMEM((L, DP), jnp.bfloat16) for _ in range(NBUF)),
            tuple(pltpu.SemaphoreType.DMA for _ in range(NBUF)),
        ],
        compiler_params=pltpu.CompilerParams(use_tc_tiling_on_sc=False,
                                            needs_layout_passes=False),
    )(x, l, table)


BT = 2048


def _head_body(rep_ref, wt_ref, b_ref, o_ref):
    r = jnp.tanh(rep_ref[...])
    o_ref[...] = (
        jnp.dot(r, wt_ref[...], preferred_element_type=jnp.float32)
        + b_ref[...]
    )


def _head(rep, wt, b2d):
    return pl.pallas_call(
        _head_body,
        out_shape=jax.ShapeDtypeStruct((B, C), jnp.float32),
        grid=(B // BT,),
        in_specs=[
            pl.BlockSpec((BT, DP), lambda i: (i, 0)),
            pl.BlockSpec((DP, C), lambda i: (0, 0)),
            pl.BlockSpec((1, C), lambda i: (0, 0)),
        ],
        out_specs=pl.BlockSpec((BT, C), lambda i: (i, 0)),
    )(rep, wt, b2d)


_PERM = (tuple(range(0, DP // 2, 2)) + tuple(range(1, DP // 2, 2))
         + tuple(range(DP // 2, DP, 2)) + tuple(range(DP // 2 + 1, DP, 2)))


def kernel(x, l, lengths, table, W, b):
    table_b = jnp.pad(table.astype(jnp.bfloat16), ((0, 0), (0, DP - D)))
    wt_p = jnp.pad(W.T, ((0, DP - D), (0, 0)))[_PERM, :]
    rep = _sc_pool(x, l, table_b)
    return _head(rep, wt_p, b.reshape(1, C))


# speedup vs baseline: 1.2048x; 1.0805x over previous
import jax
import jax.numpy as jnp
from jax import lax
from jax.experimental import pallas as pl
from jax.experimental.pallas import tpu as pltpu
from jax.experimental.pallas import tpu_sc as plsc

B, L, V, D, C = 16384, 62, 100000, 50, 5
DP = 64
NC, NS, LANES = 2, 16, 16
NW = NC * NS
SPW = B // NW


GROUPS = SPW // LANES


CHUNKS = (16, 16, 16, 14)
NBUF = 4


def _sc_pool_body(x_hbm, l_hbm, table_hbm, out_hbm,
                  x_v, l_v, rep_v, bufs, sems):
    wid = lax.axis_index("s") * NC + lax.axis_index("c")
    base = wid * SPW
    pltpu.sync_copy(x_hbm.at[pl.ds(base, SPW)], x_v)
    pltpu.sync_copy(l_hbm.at[pl.ds(base, SPW)], l_v.at[pl.ds(0, SPW)])

    def issue(s, n, b):
        for c, sz in enumerate(CHUNKS):
            idx = x_v.at[s, pl.ds(16 * c, sz)]
            dst = bufs[b].at[pl.ds(16 * c, sz)]
            if c == 0:
                pltpu.async_copy(table_hbm.at[idx], dst, sems[b])
            else:
                @pl.when(16 * c < n)
                def _():
                    pltpu.async_copy(table_hbm.at[idx], dst, sems[b])

    def drain(n, b):
        for c, sz in enumerate(CHUNKS):
            dst = bufs[b].at[pl.ds(16 * c, sz)]
            cp = pltpu.make_async_copy(table_hbm.at[x_v.at[0, pl.ds(0, sz)]],
                                       dst, sems[b])
            if c == 0:
                cp.wait()
            else:
                @pl.when(16 * c < n)
                def _(cp=cp):
                    cp.wait()

    def accumulate(s, n, inv, buf):
        def unpack2(row_lo, row_hi):
            e0, o0 = plsc.unpack(row_lo, format=plsc.PackFormat.INTERLEAVED,
                                 preferred_element_type=jnp.float32)
            e1, o1 = plsc.unpack(row_hi, format=plsc.PackFormat.INTERLEAVED,
                                 preferred_element_type=jnp.float32)
            return e0, o0, e1, o1

        def inner(j, acc):
            a0, a1, a2, a3 = acc
            lo = buf[2 * j, pl.ds(0, 2 * LANES)] \
                + buf[2 * j + 1, pl.ds(0, 2 * LANES)]
            hi = buf[2 * j, pl.ds(2 * LANES, 2 * LANES)] \
                + buf[2 * j + 1, pl.ds(2 * LANES, 2 * LANES)]
            e0, o0, e1, o1 = unpack2(lo, hi)
            return (a0 + e0, a1 + o0, a2 + e1, a3 + o1)

        zero = jnp.zeros((LANES,), jnp.float32)
        a0, a1, a2, a3 = lax.fori_loop(0, n // 2, inner,
                                       (zero, zero, zero, zero))
        odd = (n % 2) == 1
        te0, to0, te1, to1 = unpack2(buf[n - 1, pl.ds(0, 2 * LANES)],
                                     buf[n - 1, pl.ds(2 * LANES, 2 * LANES)])
        zf = jnp.zeros((LANES,), jnp.float32)
        a0 = a0 + jnp.where(odd, te0, zf)
        a1 = a1 + jnp.where(odd, to0, zf)
        a2 = a2 + jnp.where(odd, te1, zf)
        a3 = a3 + jnp.where(odd, to1, zf)
        rep_v[s, pl.ds(0, LANES)] = a0 * inv
        rep_v[s, pl.ds(LANES, LANES)] = a1 * inv
        rep_v[s, pl.ds(2 * LANES, LANES)] = a2 * inv
        rep_v[s, pl.ds(3 * LANES, LANES)] = a3 * inv

    nv0 = l_v[pl.ds(0, LANES)]
    for k in range(NBUF):
        issue(k, nv0[k], k)

    def group(g, carry):
        nv = l_v[pl.ds(g * LANES, LANES)]
        nvn = l_v[pl.ds(g * LANES + LANES, LANES)]
        inv_v = 1.0 / nv.astype(jnp.float32)
        last = g == GROUPS - 1
        for k in range(LANES):
            s = g * LANES + k
            b = k % NBUF
            drain(nv[k], b)
            accumulate(s, nv[k], inv_v[k], bufs[b])
            if k < LANES - NBUF:
                issue(s + NBUF, nv[k + NBUF], b)
            else:
                @pl.when(jnp.logical_not(last))
                def _(k=k, s=s, b=b):
                    issue(s + NBUF, nvn[k - (LANES - NBUF)], b)

        return carry

    lax.fori_loop(0, GROUPS, group, 0)
    pltpu.sync_copy(rep_v, out_hbm.at[pl.ds(base, SPW)])


def _sc_pool(x, l, table):
    return pl.kernel(
        _sc_pool_body,
        out_type=jax.ShapeDtypeStruct((B, DP), jnp.float32),
        mesh=plsc.VectorSubcoreMesh(core_axis_name="c", subcore_axis_name="s"),
        scratch_types=[
            pltpu.VMEM((SPW, L), jnp.int32),
            pltpu.VMEM((SPW + LANES,), jnp.int32),
            pltpu.VMEM((SPW, DP), jnp.float32),
            tuple(pltpu.VMEM((L, DP), jnp.bfloat16) for _ in range(NBUF)),
            tuple(pltpu.SemaphoreType.DMA for _ in range(NBUF)),
        ],
        compiler_params=pltpu.CompilerParams(use_tc_tiling_on_sc=False,
                                            needs_layout_passes=False),
    )(x, l, table)


BT = 2048


def _head_body(rep_ref, wt_ref, b_ref, o_ref):
    r = jnp.tanh(rep_ref[...])
    o_ref[...] = (
        jnp.dot(r, wt_ref[...], preferred_element_type=jnp.float32)
        + b_ref[...]
    )


def _head(rep, wt, b2d):
    return pl.pallas_call(
        _head_body,
        out_shape=jax.ShapeDtypeStruct((B, C), jnp.float32),
        grid=(B // BT,),
        in_specs=[
            pl.BlockSpec((BT, DP), lambda i: (i, 0)),
            pl.BlockSpec((DP, C), lambda i: (0, 0)),
            pl.BlockSpec((1, C), lambda i: (0, 0)),
        ],
        out_specs=pl.BlockSpec((BT, C), lambda i: (i, 0)),
    )(rep, wt, b2d)


_PERM = (tuple(range(0, DP // 2, 2)) + tuple(range(1, DP // 2, 2))
         + tuple(range(DP // 2, DP, 2)) + tuple(range(DP // 2 + 1, DP, 2)))


def kernel(x, l, lengths, table, W, b):
    table_b = jnp.pad(table.astype(jnp.bfloat16), ((0, 0), (0, DP - D)))
    wt_p = jnp.pad(W.T, ((0, DP - D), (0, 0)))[_PERM, :]
    rep = _sc_pool(x, l, table_b)
    return _head(rep, wt_p, b.reshape(1, C))
